# Initial kernel scaffold; baseline (speedup 1.0000x reference)
import jax
import jax.numpy as jnp
from jax.experimental import pallas as pl


def kernel(tokens, W_pos):
    raise NotImplementedError("write your pallas kernel here")


# speedup vs baseline: 3.0604x; 3.0604x over previous
import functools

import jax
import jax.numpy as jnp
from jax import lax
from jax.experimental import pallas as pl
from jax.experimental.pallas import tpu as pltpu
from jax.experimental.pallas import tpu_sc as plsc

_NUM_CORES = 2
_NUM_SUBCORES = 16
_NUM_WORKERS = _NUM_CORES * _NUM_SUBCORES


@functools.lru_cache(maxsize=None)
def _make_bcast_rows(b: int, p: int, d: int):
    rows_per_w = p // _NUM_WORKERS
    chunk = rows_per_w
    while chunk * d * 4 > 256 * 1024:
        chunk //= 2
    n_chunks = rows_per_w // chunk

    mesh = plsc.VectorSubcoreMesh(core_axis_name="c", subcore_axis_name="s")

    @functools.partial(
        pl.kernel,
        out_type=jax.ShapeDtypeStruct((b, p, d), jnp.float32),
        mesh=mesh,
        scratch_types=[pltpu.VMEM((chunk, d), jnp.float32)],
    )
    def bcast_rows(wpos_hbm, out_hbm, buf):
        wid = lax.axis_index("s") * _NUM_CORES + lax.axis_index("c")
        base = wid * rows_per_w
        for i in range(n_chunks):
            r0 = base + i * chunk
            pltpu.sync_copy(wpos_hbm.at[pl.ds(r0, chunk)], buf)
            for bi in range(b):
                pltpu.sync_copy(buf, out_hbm.at[bi, pl.ds(r0, chunk)])

    return bcast_rows


def kernel(tokens, W_pos):
    b, p = tokens.shape
    d = W_pos.shape[1]
    return _make_bcast_rows(b, p, d)(W_pos)
